# baseline (speedup 1.0000x reference)
import functools

import jax
import jax.numpy as jnp
from jax import lax
from jax.experimental import pallas as pl
from jax.experimental.pallas import tpu as pltpu
from jax.experimental.pallas import tpu_sc as plsc

_P = (1 << 31) - 1
_DIM = 64
_TDIM = 56
_NH = 8
_BT = 71428
_KT = 500000
_WROWS = _KT * _NH // 128
_BATCH = 16384
_NC = 2
_NS = 16
_NW = _NC * _NS
_PER_W = _BATCH // _NW
_CB = 64
_NSC = _PER_W // _CB


def _i(v):
    return jnp.int32(v)


def _mod_p_mul(x1, x0, chi, clo, out_mod):
    m31 = jnp.uint32(_P)
    a = x1 * chi
    ap = (a >> 5) + ((a & 31) << 26)
    b = x1 * clo
    bp = (b >> 21) + ((b & ((1 << 21) - 1)) << 10)
    c = x0 * chi
    cp = (c >> 15) + ((c & ((1 << 15) - 1)) << 16)
    d = x0 * clo
    s = ap + bp
    s = (s >> 31) + (s & m31)
    s = s + cp
    s = (s >> 31) + (s & m31)
    s = s + d
    s = (s >> 31) + (s & m31)
    s = jnp.where(s >= m31, s - m31, s)
    return (s % jnp.uint32(out_mod)).astype(jnp.int32)


def _dyngather(v, idx):
    dnums = lax.GatherDimensionNumbers(
        offset_dims=(), collapsed_slice_dims=(0,), start_index_map=(0,))
    return lax.gather(v, idx[:, None], dnums, slice_sizes=(1,),
                      mode=lax.GatherScatterMode.PROMISE_IN_BOUNDS)


_TROWS = 71680


def _tp_body(t_ref, o_ref):
    blk = t_ref[...]
    r = lax.broadcasted_iota(jnp.int32, (_TDIM, _TDIM), 0)
    c = lax.broadcasted_iota(jnp.int32, (_TDIM, _TDIM), 1)
    eye = jnp.where(r == c, jnp.float32(1.0), jnp.float32(0.0))
    o_ref[:, : _TDIM] = lax.dot_general(
        blk, eye, (((0,), (0,)), ((), ())),
        precision=lax.Precision.HIGHEST)


_transpose_pad = pl.pallas_call(
    _tp_body,
    grid=(_TROWS // 512,),
    in_specs=[pl.BlockSpec((_TDIM, 512),
                           lambda i: (jnp.int32(0), jnp.int32(i)))],
    out_specs=pl.BlockSpec((512, 128),
                           lambda i: (jnp.int32(i), jnp.int32(0))),
    out_shape=jax.ShapeDtypeStruct((_TROWS, 128), jnp.float32),
)


_mesh = plsc.VectorSubcoreMesh(core_axis_name="c", subcore_axis_name="s")


@functools.partial(
    pl.kernel,
    mesh=_mesh,
    out_type=jax.ShapeDtypeStruct((_BATCH, _DIM), jnp.float32),
    scratch_types=[
        pltpu.VMEM((_PER_W,), jnp.int32),
        pltpu.VMEM((9 * 16,), jnp.int32),
        pltpu.VMEM((9 * 16,), jnp.int32),
        pltpu.VMEM((_NH, _CB), jnp.int32),
        pltpu.VMEM((_CB,), jnp.int32),
        pltpu.VMEM((_CB // 16, 16), jnp.int32),
        pltpu.VMEM((_NH, _CB, 128), jnp.float32),
        pltpu.VMEM((_CB, 128), jnp.float32),
        pltpu.VMEM((_CB, _DIM), jnp.float32),
        pltpu.SemaphoreType.DMA,
    ],
)
def _sc_embed(x_hbm, table_hbm, w128_hbm, chi_hbm, clo_hbm, out_hbm,
              x_v, chi_v, clo_v, idx0, idx1, pb, rows, wrow, outb, sem):
    wid = lax.axis_index("s") * _NC + lax.axis_index("c")
    base_w = pl.multiple_of(wid * _PER_W, _PER_W)
    pltpu.sync_copy(x_hbm.at[pl.ds(base_w, _PER_W)], x_v)
    pltpu.sync_copy(chi_hbm, chi_v)
    pltpu.sync_copy(clo_hbm, clo_v)

    lane = lax.iota(jnp.int32, 16)
    shift8 = jnp.minimum(lane + 8, 15)
    is_head = lane < 8
    ohs = [jnp.where(lane == 8 + k, jnp.float32(0.125), jnp.float32(0.0))
           for k in range(_NH)]

    def subchunk(sc, carry):
        def hash16(ic, c2):
            xv = x_v[pl.ds(sc * _CB + ic * 16, 16)].astype(jnp.uint32)
            x1 = xv >> 10
            x0 = xv & 1023
            for j in range(_NH):
                chi = chi_v[pl.ds(16 * j, 16)].astype(jnp.uint32)
                clo = clo_v[pl.ds(16 * j, 16)].astype(jnp.uint32)
                idx0[_i(j), pl.ds(ic * 16, 16)] = _mod_p_mul(
                    x1, x0, chi, clo, _BT)
            chi = chi_v[pl.ds(16 * _NH, 16)].astype(jnp.uint32)
            clo = clo_v[pl.ds(16 * _NH, 16)].astype(jnp.uint32)
            h1 = _mod_p_mul(x1, x0, chi, clo, _KT)
            idx1[pl.ds(ic * 16, 16)] = h1 >> 4
            pb[ic, :] = h1 & 15
            return c2

        lax.fori_loop(_i(0), _i(_CB // 16), hash16, _i(0))

        copies = [
            pltpu.async_copy(table_hbm.at[idx0.at[_i(j)]], rows.at[_i(j)],
                             sem)
            for j in range(_NH)
        ]
        copies.append(pltpu.async_copy(w128_hbm.at[idx1], wrow, sem))
        for cp in copies:
            cp.wait()

        def combine(i, c2):
            mv = _dyngather(pb[i // 16, :], jnp.full((16,), i & 15,
                                                     jnp.int32))
            ws = [_dyngather(wrow[i, pl.ds(16 * k, 16)], mv)
                  for k in range(_NH)]
            wtail = ws[0] * ohs[0]
            for k in range(1, _NH):
                wtail = wtail + ws[k] * ohs[k]
            for c in range(3):
                acc = ws[0] * rows[_i(0), i, pl.ds(c * 16, 16)]
                for j in range(1, _NH):
                    acc = acc + ws[j] * rows[_i(j), i, pl.ds(c * 16, 16)]
                outb[i, pl.ds(c * 16, 16)] = acc
            acc = ws[0] * rows[_i(0), i, pl.ds(40, 16)]
            for j in range(1, _NH):
                acc = acc + ws[j] * rows[_i(j), i, pl.ds(40, 16)]
            tail = _dyngather(acc, shift8)
            outb[i, pl.ds(48, 16)] = jnp.where(is_head, tail, wtail)
            return c2

        lax.fori_loop(_i(0), _i(_CB), combine, _i(0))
        pltpu.sync_copy(outb, out_hbm.at[pl.ds(base_w + sc * _CB, _CB)])
        return carry

    lax.fori_loop(_i(0), _i(_NSC), subchunk, _i(0))


def kernel(x, table, weights, hash0_coeffs, hash1_coeffs):
    x32 = x.astype(jnp.int32)
    table128 = _transpose_pad(table.T)
    wT = weights.T
    w128 = wT.reshape(_NH, _WROWS, 16).transpose(1, 0, 2).reshape(_WROWS, 128)
    coeffs = jnp.concatenate([hash0_coeffs, hash1_coeffs])
    chi = jnp.broadcast_to(
        (coeffs >> 16).astype(jnp.int32)[:, None], (9, 16)).reshape(144)
    clo = jnp.broadcast_to(
        (coeffs & 0xFFFF).astype(jnp.int32)[:, None], (9, 16)).reshape(144)
    return _sc_embed(x32, table128, w128, chi, clo)


# speedup vs baseline: 1.0461x; 1.0461x over previous
import functools

import jax
import jax.numpy as jnp
from jax import lax
from jax.experimental import pallas as pl
from jax.experimental.pallas import tpu as pltpu
from jax.experimental.pallas import tpu_sc as plsc

_P = (1 << 31) - 1
_DIM = 64
_TDIM = 56
_NH = 8
_BT = 71428
_KT = 500000
_WROWS = _KT * _NH // 128
_BATCH = 16384
_NC = 2
_NS = 16
_NW = _NC * _NS
_PER_W = _BATCH // _NW
_CB = 64
_NSC = _PER_W // _CB


def _i(v):
    return jnp.int32(v)


def _mod_p_mul(x1, x0, chi, clo, out_mod):
    m31 = jnp.uint32(_P)
    a = x1 * chi
    ap = (a >> 5) + ((a & 31) << 26)
    b = x1 * clo
    bp = (b >> 21) + ((b & ((1 << 21) - 1)) << 10)
    c = x0 * chi
    cp = (c >> 15) + ((c & ((1 << 15) - 1)) << 16)
    d = x0 * clo
    s = ap + bp
    s = (s >> 31) + (s & m31)
    s = s + cp
    s = (s >> 31) + (s & m31)
    s = s + d
    s = (s >> 31) + (s & m31)
    s = jnp.where(s >= m31, s - m31, s)
    return (s % jnp.uint32(out_mod)).astype(jnp.int32)


def _dyngather(v, idx):
    dnums = lax.GatherDimensionNumbers(
        offset_dims=(), collapsed_slice_dims=(0,), start_index_map=(0,))
    return lax.gather(v, idx[:, None], dnums, slice_sizes=(1,),
                      mode=lax.GatherScatterMode.PROMISE_IN_BOUNDS)


_TROWS = 71680


def _tp_body(t_ref, o_ref):
    o_ref[:, : _TDIM] = jnp.transpose(t_ref[...])


_transpose_pad = pl.pallas_call(
    _tp_body,
    grid=(_TROWS // 512,),
    in_specs=[pl.BlockSpec((_TDIM, 512),
                           lambda i: (jnp.int32(0), jnp.int32(i)))],
    out_specs=pl.BlockSpec((512, 128),
                           lambda i: (jnp.int32(i), jnp.int32(0))),
    out_shape=jax.ShapeDtypeStruct((_TROWS, 128), jnp.float32),
)


_mesh = plsc.VectorSubcoreMesh(core_axis_name="c", subcore_axis_name="s")


@functools.partial(
    pl.kernel,
    mesh=_mesh,
    out_type=jax.ShapeDtypeStruct((_BATCH, _DIM), jnp.float32),
    scratch_types=[
        pltpu.VMEM((_PER_W,), jnp.int32),
        pltpu.VMEM((9 * 16,), jnp.int32),
        pltpu.VMEM((9 * 16,), jnp.int32),
        pltpu.VMEM((_NH, _CB), jnp.int32),
        pltpu.VMEM((_CB,), jnp.int32),
        pltpu.VMEM((_CB // 16, 16), jnp.int32),
        pltpu.VMEM((_NH, _CB, 128), jnp.float32),
        pltpu.VMEM((_CB, 128), jnp.float32),
        pltpu.VMEM((_CB, _DIM), jnp.float32),
        pltpu.SemaphoreType.DMA,
    ],
)
def _sc_embed(x_hbm, table_hbm, w128_hbm, chi_hbm, clo_hbm, out_hbm,
              x_v, chi_v, clo_v, idx0, idx1, pb, rows, wrow, outb, sem):
    wid = lax.axis_index("s") * _NC + lax.axis_index("c")
    base_w = pl.multiple_of(wid * _PER_W, _PER_W)
    pltpu.sync_copy(x_hbm.at[pl.ds(base_w, _PER_W)], x_v)
    pltpu.sync_copy(chi_hbm, chi_v)
    pltpu.sync_copy(clo_hbm, clo_v)

    lane = lax.iota(jnp.int32, 16)
    shift8 = jnp.minimum(lane + 8, 15)
    is_head = lane < 8
    ohs = [jnp.where(lane == 8 + k, jnp.float32(0.125), jnp.float32(0.0))
           for k in range(_NH)]

    def subchunk(sc, carry):
        def hash16(ic, c2):
            xv = x_v[pl.ds(sc * _CB + ic * 16, 16)].astype(jnp.uint32)
            x1 = xv >> 10
            x0 = xv & 1023
            for j in range(_NH):
                chi = chi_v[pl.ds(16 * j, 16)].astype(jnp.uint32)
                clo = clo_v[pl.ds(16 * j, 16)].astype(jnp.uint32)
                idx0[_i(j), pl.ds(ic * 16, 16)] = _mod_p_mul(
                    x1, x0, chi, clo, _BT)
            chi = chi_v[pl.ds(16 * _NH, 16)].astype(jnp.uint32)
            clo = clo_v[pl.ds(16 * _NH, 16)].astype(jnp.uint32)
            h1 = _mod_p_mul(x1, x0, chi, clo, _KT)
            idx1[pl.ds(ic * 16, 16)] = h1 >> 4
            pb[ic, :] = h1 & 15
            return c2

        lax.fori_loop(_i(0), _i(_CB // 16), hash16, _i(0))

        copies = [
            pltpu.async_copy(table_hbm.at[idx0.at[_i(j)]], rows.at[_i(j)],
                             sem)
            for j in range(_NH)
        ]
        copies.append(pltpu.async_copy(w128_hbm.at[idx1], wrow, sem))
        for cp in copies:
            cp.wait()

        def combine(i, c2):
            mv = _dyngather(pb[i // 16, :], jnp.full((16,), i & 15,
                                                     jnp.int32))
            ws = [_dyngather(wrow[i, pl.ds(16 * k, 16)], mv)
                  for k in range(_NH)]
            wtail = ws[0] * ohs[0]
            for k in range(1, _NH):
                wtail = wtail + ws[k] * ohs[k]
            for c in range(3):
                acc = ws[0] * rows[_i(0), i, pl.ds(c * 16, 16)]
                for j in range(1, _NH):
                    acc = acc + ws[j] * rows[_i(j), i, pl.ds(c * 16, 16)]
                outb[i, pl.ds(c * 16, 16)] = acc
            acc = ws[0] * rows[_i(0), i, pl.ds(40, 16)]
            for j in range(1, _NH):
                acc = acc + ws[j] * rows[_i(j), i, pl.ds(40, 16)]
            tail = _dyngather(acc, shift8)
            outb[i, pl.ds(48, 16)] = jnp.where(is_head, tail, wtail)
            return c2

        lax.fori_loop(_i(0), _i(_CB), combine, _i(0))
        pltpu.sync_copy(outb, out_hbm.at[pl.ds(base_w + sc * _CB, _CB)])
        return carry

    lax.fori_loop(_i(0), _i(_NSC), subchunk, _i(0))


def kernel(x, table, weights, hash0_coeffs, hash1_coeffs):
    x32 = x.astype(jnp.int32)
    table128 = _transpose_pad(table.T)
    wT = weights.T
    w128 = wT.reshape(_NH, _WROWS, 16).transpose(1, 0, 2).reshape(_WROWS, 128)
    coeffs = jnp.concatenate([hash0_coeffs, hash1_coeffs])
    chi = jnp.broadcast_to(
        (coeffs >> 16).astype(jnp.int32)[:, None], (9, 16)).reshape(144)
    clo = jnp.broadcast_to(
        (coeffs & 0xFFFF).astype(jnp.int32)[:, None], (9, 16)).reshape(144)
    return _sc_embed(x32, table128, w128, chi, clo)


# speedup vs baseline: 1.2504x; 1.1954x over previous
import functools

import jax
import jax.numpy as jnp
from jax import lax
from jax.experimental import pallas as pl
from jax.experimental.pallas import tpu as pltpu
from jax.experimental.pallas import tpu_sc as plsc

_P = (1 << 31) - 1
_DIM = 64
_TDIM = 56
_NH = 8
_BT = 71428
_KT = 500000
_WROWS = _KT * _NH // 128
_BATCH = 16384
_NC = 2
_NS = 16
_NW = _NC * _NS
_PER_W = _BATCH // _NW
_CB = 64
_NSC = _PER_W // _CB


def _i(v):
    return jnp.int32(v)


def _mod_p_mul(x1, x0, chi, clo, out_mod):
    m31 = jnp.uint32(_P)
    a = x1 * chi
    ap = (a >> 5) + ((a & 31) << 26)
    b = x1 * clo
    bp = (b >> 21) + ((b & ((1 << 21) - 1)) << 10)
    c = x0 * chi
    cp = (c >> 15) + ((c & ((1 << 15) - 1)) << 16)
    d = x0 * clo
    s = ap + bp
    s = (s >> 31) + (s & m31)
    s = s + cp
    s = (s >> 31) + (s & m31)
    s = s + d
    s = (s >> 31) + (s & m31)
    s = jnp.where(s >= m31, s - m31, s)
    return (s % jnp.uint32(out_mod)).astype(jnp.int32)


def _dyngather(v, idx):
    dnums = lax.GatherDimensionNumbers(
        offset_dims=(), collapsed_slice_dims=(0,), start_index_map=(0,))
    return lax.gather(v, idx[:, None], dnums, slice_sizes=(1,),
                      mode=lax.GatherScatterMode.PROMISE_IN_BOUNDS)


_TROWS = 71680


def _tp_body(t_ref, o_ref):
    o_ref[:, : _TDIM] = jnp.transpose(t_ref[...])


_transpose_pad = pl.pallas_call(
    _tp_body,
    grid=(_TROWS // 2048,),
    in_specs=[pl.BlockSpec((_TDIM, 2048),
                           lambda i: (jnp.int32(0), jnp.int32(i)))],
    out_specs=pl.BlockSpec((2048, 128),
                           lambda i: (jnp.int32(i), jnp.int32(0))),
    out_shape=jax.ShapeDtypeStruct((_TROWS, 128), jnp.float32),
)


_mesh = plsc.VectorSubcoreMesh(core_axis_name="c", subcore_axis_name="s")


@functools.partial(
    pl.kernel,
    mesh=_mesh,
    out_type=jax.ShapeDtypeStruct((_BATCH, _DIM), jnp.float32),
    scratch_types=[
        pltpu.VMEM((_PER_W,), jnp.int32),
        pltpu.VMEM((9 * 16,), jnp.int32),
        pltpu.VMEM((9 * 16,), jnp.int32),
        pltpu.VMEM((_NH, _CB), jnp.int32),
        pltpu.VMEM((_CB,), jnp.int32),
        pltpu.VMEM((_CB // 16, 16), jnp.int32),
        pltpu.VMEM((_NH, _CB, 128), jnp.float32),
        pltpu.VMEM((_CB, 128), jnp.float32),
        pltpu.VMEM((_CB, _DIM), jnp.float32),
        pltpu.SemaphoreType.DMA,
    ],
)
def _sc_embed(x_hbm, table_hbm, w128_hbm, chi_hbm, clo_hbm, out_hbm,
              x_v, chi_v, clo_v, idx0, idx1, pb, rows, wrow, outb, sem):
    wid = lax.axis_index("s") * _NC + lax.axis_index("c")
    base_w = pl.multiple_of(wid * _PER_W, _PER_W)
    pltpu.sync_copy(x_hbm.at[pl.ds(base_w, _PER_W)], x_v)
    pltpu.sync_copy(chi_hbm, chi_v)
    pltpu.sync_copy(clo_hbm, clo_v)

    lane = lax.iota(jnp.int32, 16)
    shift8 = jnp.minimum(lane + 8, 15)
    is_head = lane < 8
    ohs = [jnp.where(lane == 8 + k, jnp.float32(0.125), jnp.float32(0.0))
           for k in range(_NH)]

    def subchunk(sc, carry):
        def hash16(ic, c2):
            xv = x_v[pl.ds(sc * _CB + ic * 16, 16)].astype(jnp.uint32)
            x1 = xv >> 10
            x0 = xv & 1023
            for j in range(_NH):
                chi = chi_v[pl.ds(16 * j, 16)].astype(jnp.uint32)
                clo = clo_v[pl.ds(16 * j, 16)].astype(jnp.uint32)
                idx0[_i(j), pl.ds(ic * 16, 16)] = _mod_p_mul(
                    x1, x0, chi, clo, _BT)
            chi = chi_v[pl.ds(16 * _NH, 16)].astype(jnp.uint32)
            clo = clo_v[pl.ds(16 * _NH, 16)].astype(jnp.uint32)
            h1 = _mod_p_mul(x1, x0, chi, clo, _KT)
            idx1[pl.ds(ic * 16, 16)] = h1 >> 4
            pb[ic, :] = h1 & 15
            return c2

        lax.fori_loop(_i(0), _i(_CB // 16), hash16, _i(0))

        copies = [
            pltpu.async_copy(table_hbm.at[idx0.at[_i(j)]], rows.at[_i(j)],
                             sem)
            for j in range(_NH)
        ]
        copies.append(pltpu.async_copy(w128_hbm.at[idx1], wrow, sem))
        for cp in copies:
            cp.wait()

        def combine(i, c2):
            mv = _dyngather(pb[i // 16, :], jnp.full((16,), i & 15,
                                                     jnp.int32))
            ws = [_dyngather(wrow[i, pl.ds(16 * k, 16)], mv)
                  for k in range(_NH)]
            wtail = ws[0] * ohs[0]
            for k in range(1, _NH):
                wtail = wtail + ws[k] * ohs[k]
            for c in range(3):
                acc = ws[0] * rows[_i(0), i, pl.ds(c * 16, 16)]
                for j in range(1, _NH):
                    acc = acc + ws[j] * rows[_i(j), i, pl.ds(c * 16, 16)]
                outb[i, pl.ds(c * 16, 16)] = acc
            acc = ws[0] * rows[_i(0), i, pl.ds(40, 16)]
            for j in range(1, _NH):
                acc = acc + ws[j] * rows[_i(j), i, pl.ds(40, 16)]
            tail = _dyngather(acc, shift8)
            outb[i, pl.ds(48, 16)] = jnp.where(is_head, tail, wtail)
            return c2

        lax.fori_loop(_i(0), _i(_CB), combine, _i(0))
        pltpu.sync_copy(outb, out_hbm.at[pl.ds(base_w + sc * _CB, _CB)])
        return carry

    lax.fori_loop(_i(0), _i(_NSC), subchunk, _i(0))


def kernel(x, table, weights, hash0_coeffs, hash1_coeffs):
    x32 = x.astype(jnp.int32)
    table128 = _transpose_pad(table.T)
    wT = weights.T
    w128 = wT.reshape(_NH, _WROWS, 16).transpose(1, 0, 2).reshape(_WROWS, 128)
    coeffs = jnp.concatenate([hash0_coeffs, hash1_coeffs])
    chi = jnp.broadcast_to(
        (coeffs >> 16).astype(jnp.int32)[:, None], (9, 16)).reshape(144)
    clo = jnp.broadcast_to(
        (coeffs & 0xFFFF).astype(jnp.int32)[:, None], (9, 16)).reshape(144)
    return _sc_embed(x32, table128, w128, chi, clo)


# speedup vs baseline: 1.2510x; 1.0005x over previous
import functools

import jax
import jax.numpy as jnp
from jax import lax
from jax.experimental import pallas as pl
from jax.experimental.pallas import tpu as pltpu
from jax.experimental.pallas import tpu_sc as plsc

_P = (1 << 31) - 1
_DIM = 64
_TDIM = 56
_NH = 8
_BT = 71428
_KT = 500000
_WROWS = _KT * _NH // 128
_BATCH = 16384
_NC = 2
_NS = 16
_NW = _NC * _NS
_PER_W = _BATCH // _NW
_CB = 64
_NSC = _PER_W // _CB


def _i(v):
    return jnp.int32(v)


def _mod_p_mul(x1, x0, chi, clo, out_mod):
    m31 = jnp.uint32(_P)
    a = x1 * chi
    ap = (a >> 5) + ((a & 31) << 26)
    b = x1 * clo
    bp = (b >> 21) + ((b & ((1 << 21) - 1)) << 10)
    c = x0 * chi
    cp = (c >> 15) + ((c & ((1 << 15) - 1)) << 16)
    d = x0 * clo
    s = ap + bp
    s = (s >> 31) + (s & m31)
    s = s + cp
    s = (s >> 31) + (s & m31)
    s = s + d
    s = (s >> 31) + (s & m31)
    s = jnp.where(s >= m31, s - m31, s)
    return (s % jnp.uint32(out_mod)).astype(jnp.int32)


def _dyngather(v, idx):
    dnums = lax.GatherDimensionNumbers(
        offset_dims=(), collapsed_slice_dims=(0,), start_index_map=(0,))
    return lax.gather(v, idx[:, None], dnums, slice_sizes=(1,),
                      mode=lax.GatherScatterMode.PROMISE_IN_BOUNDS)


_TROWS = 71680


def _tp_body(t_ref, o_ref):
    o_ref[:, : _TDIM] = jnp.transpose(t_ref[...])


_transpose_pad = pl.pallas_call(
    _tp_body,
    grid=(_TROWS // 2048,),
    in_specs=[pl.BlockSpec((_TDIM, 2048),
                           lambda i: (jnp.int32(0), jnp.int32(i)))],
    out_specs=pl.BlockSpec((2048, 128),
                           lambda i: (jnp.int32(i), jnp.int32(0))),
    out_shape=jax.ShapeDtypeStruct((_TROWS, 128), jnp.float32),
)


_mesh = plsc.VectorSubcoreMesh(core_axis_name="c", subcore_axis_name="s")


@functools.partial(
    pl.kernel,
    mesh=_mesh,
    out_type=jax.ShapeDtypeStruct((_BATCH, 128), jnp.float32),
    scratch_types=[
        pltpu.VMEM((_PER_W,), jnp.int32),
        pltpu.VMEM((9 * 16,), jnp.int32),
        pltpu.VMEM((9 * 16,), jnp.int32),
        pltpu.VMEM((_NH, _CB), jnp.int32),
        pltpu.VMEM((_CB,), jnp.int32),
        pltpu.VMEM((_CB // 16, 16), jnp.int32),
        pltpu.VMEM((_NH, _CB, 128), jnp.float32),
        pltpu.VMEM((_CB, 128), jnp.float32),
        pltpu.VMEM((_CB, 128), jnp.float32),
        pltpu.SemaphoreType.DMA,
    ],
)
def _sc_embed(x_hbm, table_hbm, w128_hbm, chi_hbm, clo_hbm, out_hbm,
              x_v, chi_v, clo_v, idx0, idx1, pb, rows, wrow, outb, sem):
    wid = lax.axis_index("s") * _NC + lax.axis_index("c")
    base_w = pl.multiple_of(wid * _PER_W, _PER_W)
    pltpu.sync_copy(x_hbm.at[pl.ds(base_w, _PER_W)], x_v)
    pltpu.sync_copy(chi_hbm, chi_v)
    pltpu.sync_copy(clo_hbm, clo_v)

    lane = lax.iota(jnp.int32, 16)
    shift8 = jnp.minimum(lane + 8, 15)
    is_head = lane < 8
    ohs = [jnp.where(lane == 8 + k, jnp.float32(0.125), jnp.float32(0.0))
           for k in range(_NH)]

    def subchunk(sc, carry):
        def hash16(ic, c2):
            xv = x_v[pl.ds(sc * _CB + ic * 16, 16)].astype(jnp.uint32)
            x1 = xv >> 10
            x0 = xv & 1023
            for j in range(_NH):
                chi = chi_v[pl.ds(16 * j, 16)].astype(jnp.uint32)
                clo = clo_v[pl.ds(16 * j, 16)].astype(jnp.uint32)
                idx0[_i(j), pl.ds(ic * 16, 16)] = _mod_p_mul(
                    x1, x0, chi, clo, _BT)
            chi = chi_v[pl.ds(16 * _NH, 16)].astype(jnp.uint32)
            clo = clo_v[pl.ds(16 * _NH, 16)].astype(jnp.uint32)
            h1 = _mod_p_mul(x1, x0, chi, clo, _KT)
            idx1[pl.ds(ic * 16, 16)] = h1 >> 4
            pb[ic, :] = h1 & 15
            return c2

        lax.fori_loop(_i(0), _i(_CB // 16), hash16, _i(0))

        copies = [
            pltpu.async_copy(table_hbm.at[idx0.at[_i(j)]], rows.at[_i(j)],
                             sem)
            for j in range(_NH)
        ]
        copies.append(pltpu.async_copy(w128_hbm.at[idx1], wrow, sem))
        for cp in copies:
            cp.wait()

        def combine(i, c2):
            mv = _dyngather(pb[i // 16, :], jnp.full((16,), i & 15,
                                                     jnp.int32))
            ws = [_dyngather(wrow[i, pl.ds(16 * k, 16)], mv)
                  for k in range(_NH)]
            wtail = ws[0] * ohs[0]
            for k in range(1, _NH):
                wtail = wtail + ws[k] * ohs[k]
            for c in range(3):
                acc = ws[0] * rows[_i(0), i, pl.ds(c * 16, 16)]
                for j in range(1, _NH):
                    acc = acc + ws[j] * rows[_i(j), i, pl.ds(c * 16, 16)]
                outb[i, pl.ds(c * 16, 16)] = acc
            acc = ws[0] * rows[_i(0), i, pl.ds(40, 16)]
            for j in range(1, _NH):
                acc = acc + ws[j] * rows[_i(j), i, pl.ds(40, 16)]
            tail = _dyngather(acc, shift8)
            outb[i, pl.ds(48, 16)] = jnp.where(is_head, tail, wtail)
            return c2

        lax.fori_loop(_i(0), _i(_CB), combine, _i(0))
        pltpu.sync_copy(outb, out_hbm.at[pl.ds(base_w + sc * _CB, _CB)])
        return carry

    lax.fori_loop(_i(0), _i(_NSC), subchunk, _i(0))


def kernel(x, table, weights, hash0_coeffs, hash1_coeffs):
    x32 = x.astype(jnp.int32)
    table128 = _transpose_pad(table.T)
    wT = weights.T
    w128 = wT.reshape(_NH, _WROWS, 16).transpose(1, 0, 2).reshape(_WROWS, 128)
    coeffs = jnp.concatenate([hash0_coeffs, hash1_coeffs])
    chi = jnp.broadcast_to(
        (coeffs >> 16).astype(jnp.int32)[:, None], (9, 16)).reshape(144)
    clo = jnp.broadcast_to(
        (coeffs & 0xFFFF).astype(jnp.int32)[:, None], (9, 16)).reshape(144)
    return _sc_embed(x32, table128, w128, chi, clo)[:, :_DIM]
